# baseline (speedup 1.0000x reference)
import jax
import jax.numpy as jnp
from jax.experimental import pallas as pl
from jax.experimental.pallas import tpu as pltpu

_BM1 = 2000
_BM = 400
_NC = 2
_KC = 2048


def _kchunks(n):
    return [(k0, min(_KC, n - k0)) for k0 in range(0, n, _KC)]


def _support_body(x_ref, w1_ref, s_ref):
    xb = x_ref[...].astype(jnp.bfloat16)
    s = jax.lax.dot_general(xb, w1_ref[...], (((1,), (0,)), ((), ())),
                            preferred_element_type=jnp.float32)
    s_ref[...] = s.astype(jnp.bfloat16)


def _dotf32(a, b):
    return jax.lax.dot_general(a, b, (((1,), (0,)), ((), ())),
                               preferred_element_type=jnp.float32)


def _make_gcn_body(nb, n):
    chunks = _kchunks(n)

    def _gcn_body(adj_ref, sup_ref, w2_ref, b1_ref, b2_ref, o_ref,
                  *scratch):
        cache_ref = scratch[0] if _NC else None
        t_ref = scratch[-1]
        s = pl.program_id(0)

        @pl.when(s < nb)
        def _pass1():
            acc = jnp.zeros((_BM, sup_ref.shape[1]), jnp.float32)
            for k0, kw in chunks:
                a = adj_ref[:, k0:k0 + kw].astype(jnp.bfloat16)
                acc = acc + _dotf32(a, sup_ref[k0:k0 + kw, :])
                if _NC:
                    @pl.when(s >= nb - _NC)
                    def _stash():
                        cache_ref[pl.ds((s - (nb - _NC)) * _BM, _BM),
                                  k0:k0 + kw] = a

            h = jnp.maximum(acc + b1_ref[...], 0.0).astype(jnp.bfloat16)
            t = _dotf32(h, w2_ref[...])
            t_ref[pl.ds(s * _BM, _BM), :] = t.astype(jnp.bfloat16)

        @pl.when(s >= nb)
        def _pass2():
            i = s - nb

            if _NC:
                @pl.when(i >= nb - _NC)
                def _cached():
                    acc = jnp.zeros((_BM, t_ref.shape[1]), jnp.float32)
                    for k0, kw in chunks:
                        a = cache_ref[pl.ds((i - (nb - _NC)) * _BM, _BM),
                                      k0:k0 + kw]
                        acc = acc + _dotf32(a, t_ref[k0:k0 + kw, :])
                    o_ref[...] = jnp.maximum(acc + b2_ref[...], 0.0)

            @pl.when(i < nb - _NC)
            def _streamed():
                acc = jnp.zeros((_BM, t_ref.shape[1]), jnp.float32)
                for k0, kw in chunks:
                    a = adj_ref[:, k0:k0 + kw].astype(jnp.bfloat16)
                    acc = acc + _dotf32(a, t_ref[k0:k0 + kw, :])
                o_ref[...] = jnp.maximum(acc + b2_ref[...], 0.0)

    return _gcn_body


def kernel(x, adj, W1, b1, W2, b2):
    n, nfeat = x.shape
    nhid = W1.shape[1]
    nout = W2.shape[1]
    nb = n // _BM
    w1 = W1.astype(jnp.bfloat16)
    w2 = W2.astype(jnp.bfloat16)
    b1r = b1.reshape(1, nhid)
    b2r = b2.reshape(1, nout)

    support = pl.pallas_call(
        _support_body,
        grid=(n // _BM1,),
        in_specs=[
            pl.BlockSpec((_BM1, nfeat), lambda i: (i, 0)),
            pl.BlockSpec((nfeat, nhid), lambda i: (0, 0)),
        ],
        out_specs=pl.BlockSpec((_BM1, nhid), lambda i: (i, 0)),
        out_shape=jax.ShapeDtypeStruct((n, nhid), jnp.bfloat16),
    )(x, w1)

    def adj_imap(s):
        i = s - nb
        if _NC:
            return (jnp.where(s < nb, s,
                              jnp.where(i < nb - _NC, i, nb - _NC - 1)), 0)
        return (jnp.where(s < nb, s, i), 0)

    def out_imap(s):
        return (jnp.maximum(s - nb, 0), 0)

    out = pl.pallas_call(
        _make_gcn_body(nb, n),
        grid=(2 * nb,),
        in_specs=[
            pl.BlockSpec((_BM, n), adj_imap),
            pl.BlockSpec((n, nhid), lambda s: (0, 0)),
            pl.BlockSpec((nhid, nout), lambda s: (0, 0)),
            pl.BlockSpec((1, nhid), lambda s: (0, 0)),
            pl.BlockSpec((1, nout), lambda s: (0, 0)),
        ],
        out_specs=pl.BlockSpec((_BM, nout), out_imap),
        out_shape=jax.ShapeDtypeStruct((n, nout), jnp.float32),
        scratch_shapes=(
            ([pltpu.VMEM((_NC * _BM, n), jnp.bfloat16)] if _NC else [])
            + [pltpu.VMEM((n, nout), jnp.bfloat16)]
        ),
        compiler_params=pltpu.CompilerParams(
            vmem_limit_bytes=64 * 1024 * 1024,
        ),
    )(adj, support, w2, b1r, b2r)
    return out


# speedup vs baseline: 1.0250x; 1.0250x over previous
import jax
import jax.numpy as jnp
from jax.experimental import pallas as pl
from jax.experimental.pallas import tpu as pltpu

_BM1 = 2000
_BM = 400
_NC = 0
_KC = 4096


def _kchunks(n):
    return [(k0, min(_KC, n - k0)) for k0 in range(0, n, _KC)]


def _support_body(x_ref, w1_ref, s_ref):
    xb = x_ref[...].astype(jnp.bfloat16)
    s = jax.lax.dot_general(xb, w1_ref[...], (((1,), (0,)), ((), ())),
                            preferred_element_type=jnp.float32)
    s_ref[...] = s.astype(jnp.bfloat16)


def _dotf32(a, b):
    return jax.lax.dot_general(a, b, (((1,), (0,)), ((), ())),
                               preferred_element_type=jnp.float32)


def _make_gcn_body(nb, n):
    chunks = _kchunks(n)

    def _gcn_body(adj_ref, sup_ref, w2_ref, b1_ref, b2_ref, o_ref,
                  *scratch):
        cache_ref = scratch[0] if _NC else None
        t_ref = scratch[-1]
        s = pl.program_id(0)

        @pl.when(s < nb)
        def _pass1():
            acc = jnp.zeros((_BM, sup_ref.shape[1]), jnp.float32)
            for k0, kw in chunks:
                a = adj_ref[:, k0:k0 + kw].astype(jnp.bfloat16)
                acc = acc + _dotf32(a, sup_ref[k0:k0 + kw, :])
                if _NC:
                    @pl.when(s < _NC)
                    def _stash():
                        cache_ref[pl.ds(s * _BM, _BM), k0:k0 + kw] = a

            h = jnp.maximum(acc + b1_ref[...], 0.0).astype(jnp.bfloat16)
            t = _dotf32(h, w2_ref[...])
            t_ref[pl.ds(s * _BM, _BM), :] = t.astype(jnp.bfloat16)

        @pl.when(s >= nb)
        def _pass2():
            i = s - nb

            if _NC:
                @pl.when(i < _NC)
                def _cached():
                    acc = jnp.zeros((_BM, t_ref.shape[1]), jnp.float32)
                    for k0, kw in chunks:
                        a = cache_ref[pl.ds(i * _BM, _BM), k0:k0 + kw]
                        acc = acc + _dotf32(a, t_ref[k0:k0 + kw, :])
                    o_ref[...] = jnp.maximum(acc + b2_ref[...], 0.0)

            @pl.when(i >= _NC)
            def _streamed():
                acc = jnp.zeros((_BM, t_ref.shape[1]), jnp.float32)
                for k0, kw in chunks:
                    a = adj_ref[:, k0:k0 + kw].astype(jnp.bfloat16)
                    acc = acc + _dotf32(a, t_ref[k0:k0 + kw, :])
                o_ref[...] = jnp.maximum(acc + b2_ref[...], 0.0)

    return _gcn_body


def kernel(x, adj, W1, b1, W2, b2):
    n, nfeat = x.shape
    nhid = W1.shape[1]
    nout = W2.shape[1]
    nb = n // _BM
    w1 = W1.astype(jnp.bfloat16)
    w2 = W2.astype(jnp.bfloat16)
    b1r = b1.reshape(1, nhid)
    b2r = b2.reshape(1, nout)

    support = pl.pallas_call(
        _support_body,
        grid=(n // _BM1,),
        in_specs=[
            pl.BlockSpec((_BM1, nfeat), lambda i: (i, 0)),
            pl.BlockSpec((nfeat, nhid), lambda i: (0, 0)),
        ],
        out_specs=pl.BlockSpec((_BM1, nhid), lambda i: (i, 0)),
        out_shape=jax.ShapeDtypeStruct((n, nhid), jnp.bfloat16),
    )(x, w1)

    def adj_imap(s):
        i = s - nb
        if _NC:
            return (jnp.where(s < nb, s, jnp.where(i >= _NC, i, nb - 1)), 0)
        return (jnp.where(s < nb, s, i), 0)

    def out_imap(s):
        return (jnp.maximum(s - nb, 0), 0)

    out = pl.pallas_call(
        _make_gcn_body(nb, n),
        grid=(2 * nb,),
        in_specs=[
            pl.BlockSpec((_BM, n), adj_imap),
            pl.BlockSpec((n, nhid), lambda s: (0, 0)),
            pl.BlockSpec((nhid, nout), lambda s: (0, 0)),
            pl.BlockSpec((1, nhid), lambda s: (0, 0)),
            pl.BlockSpec((1, nout), lambda s: (0, 0)),
        ],
        out_specs=pl.BlockSpec((_BM, nout), out_imap),
        out_shape=jax.ShapeDtypeStruct((n, nout), jnp.float32),
        scratch_shapes=(
            ([pltpu.VMEM((_NC * _BM, n), jnp.bfloat16)] if _NC else [])
            + [pltpu.VMEM((n, nout), jnp.bfloat16)]
        ),
        compiler_params=pltpu.CompilerParams(
            vmem_limit_bytes=64 * 1024 * 1024,
        ),
    )(adj, support, w2, b1r, b2r)
    return out


# speedup vs baseline: 1.0299x; 1.0048x over previous
import jax
import jax.numpy as jnp
from jax.experimental import pallas as pl
from jax.experimental.pallas import tpu as pltpu

_BM1 = 2000
_BM = 400
_NC = 0
_KC = 2048


def _kchunks(n):
    return [(k0, min(_KC, n - k0)) for k0 in range(0, n, _KC)]


def _support_body(x_ref, w1_ref, s_ref):
    xb = x_ref[...].astype(jnp.bfloat16)
    s = jax.lax.dot_general(xb, w1_ref[...], (((1,), (0,)), ((), ())),
                            preferred_element_type=jnp.float32)
    s_ref[...] = s.astype(jnp.bfloat16)


def _dotf32(a, b):
    return jax.lax.dot_general(a, b, (((1,), (0,)), ((), ())),
                               preferred_element_type=jnp.float32)


def _make_gcn_body(nb, n):
    chunks = _kchunks(n)

    def _gcn_body(adj_ref, sup_ref, w2_ref, b1_ref, b2_ref, o_ref,
                  *scratch):
        cache_ref = scratch[0] if _NC else None
        t_ref = scratch[-1]
        s = pl.program_id(0)

        @pl.when(s < nb)
        def _pass1():
            acc = jnp.zeros((_BM, sup_ref.shape[1]), jnp.float32)
            for k0, kw in chunks:
                a = adj_ref[:, k0:k0 + kw].astype(jnp.bfloat16)
                acc = acc + _dotf32(a, sup_ref[k0:k0 + kw, :])
                if _NC:
                    @pl.when(s < _NC)
                    def _stash():
                        cache_ref[pl.ds(s * _BM, _BM), k0:k0 + kw] = a

            h = jnp.maximum(acc + b1_ref[...], 0.0).astype(jnp.bfloat16)
            t = _dotf32(h, w2_ref[...])
            t_ref[pl.ds(s * _BM, _BM), :] = t.astype(jnp.bfloat16)

        @pl.when(s >= nb)
        def _pass2():
            i = s - nb

            if _NC:
                @pl.when(i < _NC)
                def _cached():
                    acc = jnp.zeros((_BM, t_ref.shape[1]), jnp.float32)
                    for k0, kw in chunks:
                        a = cache_ref[pl.ds(i * _BM, _BM), k0:k0 + kw]
                        acc = acc + _dotf32(a, t_ref[k0:k0 + kw, :])
                    o_ref[...] = jnp.maximum(acc + b2_ref[...], 0.0)

            @pl.when(i >= _NC)
            def _streamed():
                acc = jnp.zeros((_BM, t_ref.shape[1]), jnp.float32)
                for k0, kw in chunks:
                    a = adj_ref[:, k0:k0 + kw].astype(jnp.bfloat16)
                    acc = acc + _dotf32(a, t_ref[k0:k0 + kw, :])
                o_ref[...] = jnp.maximum(acc + b2_ref[...], 0.0)

    return _gcn_body


def kernel(x, adj, W1, b1, W2, b2):
    n, nfeat = x.shape
    nhid = W1.shape[1]
    nout = W2.shape[1]
    nb = n // _BM
    w1 = W1.astype(jnp.bfloat16)
    w2 = W2.astype(jnp.bfloat16)
    b1r = b1.reshape(1, nhid)
    b2r = b2.reshape(1, nout)

    support = pl.pallas_call(
        _support_body,
        grid=(n // _BM1,),
        in_specs=[
            pl.BlockSpec((_BM1, nfeat), lambda i: (i, 0)),
            pl.BlockSpec((nfeat, nhid), lambda i: (0, 0)),
        ],
        out_specs=pl.BlockSpec((_BM1, nhid), lambda i: (i, 0)),
        out_shape=jax.ShapeDtypeStruct((n, nhid), jnp.bfloat16),
    )(x, w1)

    def adj_imap(s):
        i = s - nb
        if _NC:
            return (jnp.where(s < nb, s, jnp.where(i >= _NC, i, nb - 1)), 0)
        return (jnp.where(s < nb, s, i), 0)

    def out_imap(s):
        return (jnp.maximum(s - nb, 0), 0)

    out = pl.pallas_call(
        _make_gcn_body(nb, n),
        grid=(2 * nb,),
        in_specs=[
            pl.BlockSpec((_BM, n), adj_imap),
            pl.BlockSpec((n, nhid), lambda s: (0, 0)),
            pl.BlockSpec((nhid, nout), lambda s: (0, 0)),
            pl.BlockSpec((1, nhid), lambda s: (0, 0)),
            pl.BlockSpec((1, nout), lambda s: (0, 0)),
        ],
        out_specs=pl.BlockSpec((_BM, nout), out_imap),
        out_shape=jax.ShapeDtypeStruct((n, nout), jnp.float32),
        scratch_shapes=(
            ([pltpu.VMEM((_NC * _BM, n), jnp.bfloat16)] if _NC else [])
            + [pltpu.VMEM((n, nout), jnp.bfloat16)]
        ),
        compiler_params=pltpu.CompilerParams(
            vmem_limit_bytes=64 * 1024 * 1024,
        ),
    )(adj, support, w2, b1r, b2r)
    return out


# speedup vs baseline: 1.0421x; 1.0119x over previous
import jax
import jax.numpy as jnp
from jax.experimental import pallas as pl
from jax.experimental.pallas import tpu as pltpu

_BM1 = 2000
_BM = 400
_KC = 2048


def _kchunks(n):
    return [(k0, min(_KC, n - k0)) for k0 in range(0, n, _KC)]


def _dotf32(a, b):
    return jax.lax.dot_general(a, b, (((1,), (0,)), ((), ())),
                               preferred_element_type=jnp.float32)


def _make_fused_body(ns, nb, n):
    chunks = _kchunks(n)

    def _body(x_ref, adj_ref, w1_ref, w2_ref, b1_ref, b2_ref, o_ref,
              sup_ref, t_ref):
        u = pl.program_id(0)

        @pl.when(u < ns)
        def _support():
            xb = x_ref[...].astype(jnp.bfloat16)
            sup = _dotf32(xb, w1_ref[...])
            sup_ref[pl.ds(u * _BM1, _BM1), :] = sup.astype(jnp.bfloat16)

        @pl.when(jnp.logical_and(u >= ns, u < ns + nb))
        def _pass1():
            s = u - ns
            acc = jnp.zeros((_BM, sup_ref.shape[1]), jnp.float32)
            for k0, kw in chunks:
                a = adj_ref[:, k0:k0 + kw].astype(jnp.bfloat16)
                acc = acc + _dotf32(a, sup_ref[k0:k0 + kw, :])
            h = jnp.maximum(acc + b1_ref[...], 0.0).astype(jnp.bfloat16)
            t = _dotf32(h, w2_ref[...])
            t_ref[pl.ds(s * _BM, _BM), :] = t.astype(jnp.bfloat16)

        @pl.when(u >= ns + nb)
        def _pass2():
            acc = jnp.zeros((_BM, t_ref.shape[1]), jnp.float32)
            for k0, kw in chunks:
                a = adj_ref[:, k0:k0 + kw].astype(jnp.bfloat16)
                acc = acc + _dotf32(a, t_ref[k0:k0 + kw, :])
            o_ref[...] = jnp.maximum(acc + b2_ref[...], 0.0)

    return _body


def kernel(x, adj, W1, b1, W2, b2):
    n, nfeat = x.shape
    nhid = W1.shape[1]
    nout = W2.shape[1]
    ns = n // _BM1
    nb = n // _BM
    w1 = W1.astype(jnp.bfloat16)
    w2 = W2.astype(jnp.bfloat16)
    b1r = b1.reshape(1, nhid)
    b2r = b2.reshape(1, nout)

    def x_imap(u):
        return (jnp.minimum(u, ns - 1), 0)

    def adj_imap(u):
        return (jnp.where(u < ns + nb, jnp.maximum(u - ns, 0),
                          u - ns - nb), 0)

    def out_imap(u):
        return (jnp.maximum(u - ns - nb, 0), 0)

    out = pl.pallas_call(
        _make_fused_body(ns, nb, n),
        grid=(ns + 2 * nb,),
        in_specs=[
            pl.BlockSpec((_BM1, nfeat), x_imap),
            pl.BlockSpec((_BM, n), adj_imap),
            pl.BlockSpec((nfeat, nhid), lambda u: (0, 0)),
            pl.BlockSpec((nhid, nout), lambda u: (0, 0)),
            pl.BlockSpec((1, nhid), lambda u: (0, 0)),
            pl.BlockSpec((1, nout), lambda u: (0, 0)),
        ],
        out_specs=pl.BlockSpec((_BM, nout), out_imap),
        out_shape=jax.ShapeDtypeStruct((n, nout), jnp.float32),
        scratch_shapes=[
            pltpu.VMEM((n, nhid), jnp.bfloat16),
            pltpu.VMEM((n, nout), jnp.bfloat16),
        ],
        compiler_params=pltpu.CompilerParams(
            vmem_limit_bytes=64 * 1024 * 1024,
        ),
    )(x, adj, w1, w2, b1r, b2r)
    return out
